# baseline (speedup 1.0000x reference)
import functools

import jax
import jax.numpy as jnp
from jax import lax
from jax.experimental import pallas as pl
from jax.experimental.pallas import tpu as pltpu
from jax.experimental.pallas import tpu_sc as plsc


def _make_sc_broadcast(S, N, D, dtype):
    info = plsc.get_sparse_core_info()
    num_workers = info.num_cores * info.num_subcores
    rows_per_w = S // num_workers
    chunk = 8
    g = chunk * N
    n_chunks = rows_per_w // chunk
    mesh = plsc.VectorSubcoreMesh(core_axis_name="c", subcore_axis_name="s")

    @functools.partial(
        pl.kernel,
        mesh=mesh,
        out_type=jax.ShapeDtypeStruct((S * N, D), dtype),
        scratch_types=[
            pltpu.VMEM((g, D), dtype),
            pltpu.VMEM((g, D), dtype),
            pltpu.VMEM((g,), jnp.int32),
            pltpu.VMEM((g,), jnp.int32),
            pltpu.SemaphoreType.DMA,
            pltpu.SemaphoreType.DMA,
            pltpu.SemaphoreType.DMA,
            pltpu.SemaphoreType.DMA,
        ],
    )
    def sc_kernel(table_hbm, out_hbm, buf0, buf1, idx0, idx1, gs0, gs1, ws0, ws1):
        wid = lax.axis_index("s") * info.num_cores + lax.axis_index("c")
        base = wid * rows_per_w
        bufs, idxs, gsems, wsems = [buf0, buf1], [idx0, idx1], [gs0, gs1], [ws0, ws1]

        ids = lax.iota(jnp.int32, 16)
        shift = N.bit_length() - 1
        pats = [lax.shift_right_logical(ids + 16 * k, shift) for k in range(g // 16)]

        def fire_gather(i):
            b = i % 2
            r0 = lax.broadcast_in_dim(base + i * chunk, (16,), ())
            for k, p in enumerate(pats):
                idxs[b][pl.ds(16 * k, 16)] = p + r0
            return pltpu.async_copy(table_hbm.at[idxs[b]], bufs[b], gsems[b])

        gathers = {0: fire_gather(0), 1: fire_gather(1)}
        tail = []
        for i in range(n_chunks):
            b = i % 2
            gathers[i].wait()
            w = pltpu.async_copy(
                bufs[b], out_hbm.at[pl.ds((base + i * chunk) * N, g)], wsems[b]
            )
            if i + 2 < n_chunks:
                w.wait()
                gathers[i + 2] = fire_gather(i + 2)
            else:
                tail.append(w)
        for w in tail:
            w.wait()

    return sc_kernel


def kernel(x, pos_embedding):
    S, N = x.shape
    _, D = pos_embedding.shape
    out = _make_sc_broadcast(S, N, D, pos_embedding.dtype)(pos_embedding)
    return out.reshape(S, N, D)


# speedup vs baseline: 3.4458x; 3.4458x over previous
import functools

import jax
import jax.numpy as jnp
from jax import lax
from jax.experimental import pallas as pl
from jax.experimental.pallas import tpu as pltpu
from jax.experimental.pallas import tpu_sc as plsc


def _make_sc_broadcast(S, N, D, dtype):
    info = plsc.get_sparse_core_info()
    num_workers = info.num_cores * info.num_subcores
    rows_per_w = S // num_workers
    chunk = min(32, rows_per_w)
    n_chunks = rows_per_w // chunk
    mesh = plsc.VectorSubcoreMesh(core_axis_name="c", subcore_axis_name="s")

    @functools.partial(
        pl.kernel,
        mesh=mesh,
        out_type=jax.ShapeDtypeStruct((S, N, D), dtype),
        scratch_types=[
            pltpu.VMEM((chunk, D), dtype),
            pltpu.VMEM((chunk, D), dtype),
            pltpu.SemaphoreType.DMA,
            pltpu.SemaphoreType.DMA,
            pltpu.SemaphoreType.DMA,
            pltpu.SemaphoreType.DMA,
        ],
    )
    def sc_kernel(table_hbm, out_hbm, buf0, buf1, rsem0, rsem1, wsem0, wsem1):
        wid = lax.axis_index("s") * info.num_cores + lax.axis_index("c")
        base = wid * rows_per_w
        bufs, rsems, wsems = [buf0, buf1], [rsem0, rsem1], [wsem0, wsem1]

        def src(i):
            return table_hbm.at[pl.ds(base + i * chunk, chunk)]

        reads = {
            0: pltpu.async_copy(src(0), buf0, rsem0),
            1: pltpu.async_copy(src(1), buf1, rsem1),
        }
        tail_writes = []
        for i in range(n_chunks):
            b = i % 2
            reads[i].wait()
            writes = [
                pltpu.async_copy(
                    bufs[b], out_hbm.at[pl.ds(base + i * chunk, chunk), n], wsems[b]
                )
                for n in range(N)
            ]
            if i + 2 < n_chunks:
                for h in writes:
                    h.wait()
                reads[i + 2] = pltpu.async_copy(src(i + 2), bufs[b], rsems[b])
            else:
                tail_writes.extend(writes)
        for h in tail_writes:
            h.wait()

    return sc_kernel


def kernel(x, pos_embedding):
    S, N = x.shape
    _, D = pos_embedding.shape
    return _make_sc_broadcast(S, N, D, pos_embedding.dtype)(pos_embedding)
